# baseline (speedup 1.0000x reference)
import functools

import jax
import jax.numpy as jnp
from jax import lax
from jax.experimental import pallas as pl
from jax.experimental.pallas import tpu as pltpu
from jax.experimental.pallas import tpu_sc as plsc

B = 16384
L = 50
DIM = 128
NUM_CLASS = 1000

NC = 2
NS = 16
NW = NC * NS

NBUF = 4
FLUSH_T = 4

BM = 512
CH = 4
BC = B // CH


def _sc_pool(idx2, table):
    g_per_w = BC // NW
    nt = g_per_w // NBUF
    mesh = plsc.VectorSubcoreMesh(core_axis_name="c", subcore_axis_name="s")

    @functools.partial(
        pl.kernel,
        out_type=jax.ShapeDtypeStruct((BC, 2 * DIM), jnp.float32),
        mesh=mesh,
        scratch_types=[
            pltpu.VMEM((g_per_w, 2 * L), jnp.int32),
            pltpu.VMEM((NBUF, 2 * L, DIM), jnp.float32),
            pltpu.VMEM((NBUF * FLUSH_T, 2 * DIM), jnp.float32),
            pltpu.SemaphoreType.DMA((NBUF,)),
        ],
    )
    def sc_pool(idx_hbm, table_hbm, out_hbm, idx_v, bufs, out_v, sems):
        wid = lax.axis_index("s") * NC + lax.axis_index("c")
        g0 = wid * g_per_w
        row0 = wid * g_per_w

        pltpu.sync_copy(idx_hbm.at[pl.ds(g0, g_per_w)], idx_v)

        def fire(g, b):
            pltpu.make_async_copy(
                table_hbm.at[idx_v.at[g]], bufs.at[b], sems.at[b]
            ).start()

        def drain(g, b):
            pltpu.make_async_copy(
                table_hbm.at[idx_v.at[g]], bufs.at[b], sems.at[b]
            ).wait()

        for b in range(NBUF):
            fire(b, b)

        def outer(t, carry):
            for b in range(NBUF):
                g = t * NBUF + b
                drain(g, b)
                lr = (t % FLUSH_T) * NBUF + b
                for j in range(2):
                    def body(r, accs):
                        row = j * L + 2 * r
                        return tuple(
                            accs[k]
                            + bufs[b, row, pl.ds(k * 16, 16)]
                            + bufs[b, row + 1, pl.ds(k * 16, 16)]
                            for k in range(8)
                        )

                    accs = lax.fori_loop(
                        0, L // 2, body,
                        tuple(jnp.zeros((16,), jnp.float32) for _ in range(8)),
                    )
                    for k in range(8):
                        out_v[lr, pl.ds(j * DIM + k * 16, 16)] = accs[k]

                @pl.when(t + 1 < nt)
                def _():
                    fire(g + NBUF, b)

            @pl.when(t % FLUSH_T == FLUSH_T - 1)
            def _():
                base = pl.multiple_of(
                    row0 + (t // FLUSH_T) * (NBUF * FLUSH_T), NBUF * FLUSH_T)
                pltpu.sync_copy(
                    out_v, out_hbm.at[pl.ds(base, NBUF * FLUSH_T)]
                )

            return carry

        lax.fori_loop(0, nt, outer, 0)

    return sc_pool(idx2, table)


def _mlp_body(*refs):
    (pooled_ref, head_ref, tail_ref, lab_ref, W1_ref, b1_ref,
     W2_ref, b2_ref) = refs[:8]
    logits_ref, loss_ref = refs[-2:]
    i = pl.program_id(0)
    pooled = pooled_ref[...]
    hc = jnp.sum((head_ref[...] != 0).astype(jnp.int32), axis=1,
                 keepdims=True).astype(jnp.float32)
    tc = jnp.sum((tail_ref[...] != 0).astype(jnp.int32), axis=1,
                 keepdims=True).astype(jnp.float32)
    x = jnp.concatenate([pooled[:, :DIM] / hc, pooled[:, DIM:] / tc],
                        axis=1)
    h = jnp.maximum(
        jnp.dot(x, W1_ref[...], preferred_element_type=jnp.float32)
        + b1_ref[...], 0.0)
    logits = jnp.dot(h, W2_ref[...],
                     preferred_element_type=jnp.float32) + b2_ref[...]
    logits_ref[...] = logits
    m = jnp.max(logits, axis=1, keepdims=True)
    lse = jnp.log(jnp.sum(jnp.exp(logits - m), axis=1, keepdims=True)) + m
    cls = lax.broadcasted_iota(jnp.int32, logits.shape, 1)
    picked = jnp.sum(jnp.where(cls == lab_ref[...], logits, 0.0), axis=1,
                     keepdims=True)
    nll = lse - picked

    @pl.when(i == 0)
    def _():
        loss_ref[...] = jnp.zeros_like(loss_ref)

    loss_ref[...] += (jnp.sum(nll) * (1.0 / B)).reshape(1, 1)


def _mlp(pooled, head, tail, labels_col, W1, b1, W2, b2, base, prev=None):
    grid = (BC // BM,)
    in_specs = [
        pl.BlockSpec((BM, 2 * DIM), lambda i: (i, 0)),
        pl.BlockSpec((BM, L), lambda i: (i, 0)),
        pl.BlockSpec((BM, L), lambda i: (i, 0)),
        pl.BlockSpec((BM, 1), lambda i: (i, 0)),
        pl.BlockSpec((2 * DIM, DIM), lambda i: (0, 0)),
        pl.BlockSpec((1, DIM), lambda i: (0, 0)),
        pl.BlockSpec((DIM, NUM_CLASS), lambda i: (0, 0)),
        pl.BlockSpec((1, NUM_CLASS), lambda i: (0, 0)),
    ]
    args = [pooled, head, tail, labels_col, W1, b1, W2, b2]
    aliases = {}
    if prev is not None:
        in_specs.append(pl.BlockSpec((BM, NUM_CLASS), lambda i: (0, 0)))
        args.append(prev)
        aliases = {8: 0}
    return pl.pallas_call(
        _mlp_body,
        grid=grid,
        in_specs=in_specs,
        out_specs=[
            pl.BlockSpec((BM, NUM_CLASS), lambda i, base=base: (base + i, 0)),
            pl.BlockSpec((1, 1), lambda i: (0, 0)),
        ],
        out_shape=[
            jax.ShapeDtypeStruct((B, NUM_CLASS), jnp.float32),
            jax.ShapeDtypeStruct((1, 1), jnp.float32),
        ],
        input_output_aliases=aliases,
    )(*args)


def kernel(head, tail, labels, emb_table, W1, b1, W2, b2):
    head = head.astype(jnp.int32)
    tail = tail.astype(jnp.int32)
    labels_col = labels.reshape(B, 1)
    b1r = b1.reshape(1, DIM)
    b2r = b2.reshape(1, NUM_CLASS)
    idx2 = jnp.concatenate([head, tail], axis=1)

    pooled = [_sc_pool(idx2[c * BC:(c + 1) * BC], emb_table)
              for c in range(CH)]
    logits = None
    loss = jnp.float32(0.0)
    for c in range(CH):
        sl = slice(c * BC, (c + 1) * BC)
        logits, loss_c = _mlp(pooled[c], head[sl], tail[sl], labels_col[sl],
                              W1, b1r, W2, b2r, base=c * (BC // BM),
                              prev=logits)
        loss = loss + loss_c[0, 0]
    return logits, loss


# speedup vs baseline: 1.0272x; 1.0272x over previous
import functools

import jax
import jax.numpy as jnp
from jax import lax
from jax.experimental import pallas as pl
from jax.experimental.pallas import tpu as pltpu
from jax.experimental.pallas import tpu_sc as plsc

B = 16384
L = 50
DIM = 128
NUM_CLASS = 1000

NC = 2
NS = 16
NW = NC * NS

NBUF = 4
FLUSH_T = 4

BM = 512
CH = 1
BC = B // CH


def _sc_pool(idx2, table):
    g_per_w = BC // NW
    nt = g_per_w // NBUF
    mesh = plsc.VectorSubcoreMesh(core_axis_name="c", subcore_axis_name="s")

    @functools.partial(
        pl.kernel,
        out_type=jax.ShapeDtypeStruct((BC, 2 * DIM), jnp.float32),
        mesh=mesh,
        scratch_types=[
            pltpu.VMEM((g_per_w, 2 * L), jnp.int32),
            pltpu.VMEM((NBUF, 2 * L, DIM), jnp.float32),
            pltpu.VMEM((NBUF * FLUSH_T, 2 * DIM), jnp.float32),
            pltpu.SemaphoreType.DMA((NBUF,)),
        ],
    )
    def sc_pool(idx_hbm, table_hbm, out_hbm, idx_v, bufs, out_v, sems):
        wid = lax.axis_index("s") * NC + lax.axis_index("c")
        g0 = wid * g_per_w
        row0 = wid * g_per_w

        pltpu.sync_copy(idx_hbm.at[pl.ds(g0, g_per_w)], idx_v)

        def fire(g, b):
            pltpu.make_async_copy(
                table_hbm.at[idx_v.at[g]], bufs.at[b], sems.at[b]
            ).start()

        def drain(g, b):
            pltpu.make_async_copy(
                table_hbm.at[idx_v.at[g]], bufs.at[b], sems.at[b]
            ).wait()

        for b in range(NBUF):
            fire(b, b)

        def outer(t, carry):
            for b in range(NBUF):
                g = t * NBUF + b
                drain(g, b)
                lr = (t % FLUSH_T) * NBUF + b
                for j in range(2):
                    def body(r, accs):
                        row = j * L + 2 * r
                        return tuple(
                            accs[k]
                            + bufs[b, row, pl.ds(k * 16, 16)]
                            + bufs[b, row + 1, pl.ds(k * 16, 16)]
                            for k in range(8)
                        )

                    accs = lax.fori_loop(
                        0, L // 2, body,
                        tuple(jnp.zeros((16,), jnp.float32) for _ in range(8)),
                    )
                    for k in range(8):
                        out_v[lr, pl.ds(j * DIM + k * 16, 16)] = accs[k]

                @pl.when(t + 1 < nt)
                def _():
                    fire(g + NBUF, b)

            @pl.when(t % FLUSH_T == FLUSH_T - 1)
            def _():
                base = pl.multiple_of(
                    row0 + (t // FLUSH_T) * (NBUF * FLUSH_T), NBUF * FLUSH_T)
                pltpu.sync_copy(
                    out_v, out_hbm.at[pl.ds(base, NBUF * FLUSH_T)]
                )

            return carry

        lax.fori_loop(0, nt, outer, 0)

    return sc_pool(idx2, table)


def _mlp_body(*refs):
    (pooled_ref, head_ref, tail_ref, lab_ref, W1_ref, b1_ref,
     W2_ref, b2_ref) = refs[:8]
    logits_ref, loss_ref = refs[-2:]
    i = pl.program_id(0)
    pooled = pooled_ref[...]
    hc = jnp.sum((head_ref[...] != 0).astype(jnp.int32), axis=1,
                 keepdims=True).astype(jnp.float32)
    tc = jnp.sum((tail_ref[...] != 0).astype(jnp.int32), axis=1,
                 keepdims=True).astype(jnp.float32)
    x = jnp.concatenate([pooled[:, :DIM] / hc, pooled[:, DIM:] / tc],
                        axis=1)
    h = jnp.maximum(
        jnp.dot(x, W1_ref[...], preferred_element_type=jnp.float32)
        + b1_ref[...], 0.0)
    logits = jnp.dot(h, W2_ref[...],
                     preferred_element_type=jnp.float32) + b2_ref[...]
    logits_ref[...] = logits
    m = jnp.max(logits, axis=1, keepdims=True)
    lse = jnp.log(jnp.sum(jnp.exp(logits - m), axis=1, keepdims=True)) + m
    cls = lax.broadcasted_iota(jnp.int32, logits.shape, 1)
    picked = jnp.sum(jnp.where(cls == lab_ref[...], logits, 0.0), axis=1,
                     keepdims=True)
    nll = lse - picked

    @pl.when(i == 0)
    def _():
        loss_ref[...] = jnp.zeros_like(loss_ref)

    loss_ref[...] += (jnp.sum(nll) * (1.0 / B)).reshape(1, 1)


def _mlp(pooled, head, tail, labels_col, W1, b1, W2, b2, base, prev=None):
    grid = (BC // BM,)
    in_specs = [
        pl.BlockSpec((BM, 2 * DIM), lambda i: (i, 0)),
        pl.BlockSpec((BM, L), lambda i: (i, 0)),
        pl.BlockSpec((BM, L), lambda i: (i, 0)),
        pl.BlockSpec((BM, 1), lambda i: (i, 0)),
        pl.BlockSpec((2 * DIM, DIM), lambda i: (0, 0)),
        pl.BlockSpec((1, DIM), lambda i: (0, 0)),
        pl.BlockSpec((DIM, NUM_CLASS), lambda i: (0, 0)),
        pl.BlockSpec((1, NUM_CLASS), lambda i: (0, 0)),
    ]
    args = [pooled, head, tail, labels_col, W1, b1, W2, b2]
    aliases = {}
    if prev is not None:
        in_specs.append(pl.BlockSpec((BM, NUM_CLASS), lambda i: (0, 0)))
        args.append(prev)
        aliases = {8: 0}
    return pl.pallas_call(
        _mlp_body,
        grid=grid,
        in_specs=in_specs,
        out_specs=[
            pl.BlockSpec((BM, NUM_CLASS), lambda i, base=base: (base + i, 0)),
            pl.BlockSpec((1, 1), lambda i: (0, 0)),
        ],
        out_shape=[
            jax.ShapeDtypeStruct((B, NUM_CLASS), jnp.float32),
            jax.ShapeDtypeStruct((1, 1), jnp.float32),
        ],
        input_output_aliases=aliases,
    )(*args)


def kernel(head, tail, labels, emb_table, W1, b1, W2, b2):
    head = head.astype(jnp.int32)
    tail = tail.astype(jnp.int32)
    labels_col = labels.reshape(B, 1)
    b1r = b1.reshape(1, DIM)
    b2r = b2.reshape(1, NUM_CLASS)
    idx2 = jnp.concatenate([head, tail], axis=1)

    pooled = [_sc_pool(idx2[c * BC:(c + 1) * BC], emb_table)
              for c in range(CH)]
    logits = None
    loss = jnp.float32(0.0)
    for c in range(CH):
        sl = slice(c * BC, (c + 1) * BC)
        logits, loss_c = _mlp(pooled[c], head[sl], tail[sl], labels_col[sl],
                              W1, b1r, W2, b2r, base=c * (BC // BM),
                              prev=logits)
        loss = loss + loss_c[0, 0]
    return logits, loss


# speedup vs baseline: 1.0557x; 1.0278x over previous
import functools

import jax
import jax.numpy as jnp
from jax import lax
from jax.experimental import pallas as pl
from jax.experimental.pallas import tpu as pltpu
from jax.experimental.pallas import tpu_sc as plsc

B = 16384
L = 50
DIM = 128
NUM_CLASS = 1000

NC = 2
NS = 16
NW = NC * NS

NBUF = 4
FLUSH_T = 4

BM = 1024
CH = 1
BC = B // CH


def _sc_pool(idx2, table):
    g_per_w = BC // NW
    nt = g_per_w // NBUF
    mesh = plsc.VectorSubcoreMesh(core_axis_name="c", subcore_axis_name="s")

    @functools.partial(
        pl.kernel,
        out_type=jax.ShapeDtypeStruct((BC, 2 * DIM), jnp.float32),
        mesh=mesh,
        scratch_types=[
            pltpu.VMEM((g_per_w, 2 * L), jnp.int32),
            pltpu.VMEM((NBUF, 2 * L, DIM), jnp.float32),
            pltpu.VMEM((NBUF * FLUSH_T, 2 * DIM), jnp.float32),
            pltpu.SemaphoreType.DMA((NBUF,)),
        ],
    )
    def sc_pool(idx_hbm, table_hbm, out_hbm, idx_v, bufs, out_v, sems):
        wid = lax.axis_index("s") * NC + lax.axis_index("c")
        g0 = wid * g_per_w
        row0 = wid * g_per_w

        pltpu.sync_copy(idx_hbm.at[pl.ds(g0, g_per_w)], idx_v)

        def fire(g, b):
            pltpu.make_async_copy(
                table_hbm.at[idx_v.at[g]], bufs.at[b], sems.at[b]
            ).start()

        def drain(g, b):
            pltpu.make_async_copy(
                table_hbm.at[idx_v.at[g]], bufs.at[b], sems.at[b]
            ).wait()

        for b in range(NBUF):
            fire(b, b)

        def outer(t, carry):
            for b in range(NBUF):
                g = t * NBUF + b
                drain(g, b)
                lr = (t % FLUSH_T) * NBUF + b
                for j in range(2):
                    def body(r, accs):
                        row = j * L + 2 * r
                        return tuple(
                            accs[k]
                            + bufs[b, row, pl.ds(k * 16, 16)]
                            + bufs[b, row + 1, pl.ds(k * 16, 16)]
                            for k in range(8)
                        )

                    accs = lax.fori_loop(
                        0, L // 2, body,
                        tuple(jnp.zeros((16,), jnp.float32) for _ in range(8)),
                    )
                    for k in range(8):
                        out_v[lr, pl.ds(j * DIM + k * 16, 16)] = accs[k]

                @pl.when(t + 1 < nt)
                def _():
                    fire(g + NBUF, b)

            @pl.when(t % FLUSH_T == FLUSH_T - 1)
            def _():
                base = pl.multiple_of(
                    row0 + (t // FLUSH_T) * (NBUF * FLUSH_T), NBUF * FLUSH_T)
                pltpu.sync_copy(
                    out_v, out_hbm.at[pl.ds(base, NBUF * FLUSH_T)]
                )

            return carry

        lax.fori_loop(0, nt, outer, 0)

    return sc_pool(idx2, table)


def _mlp_body(*refs):
    (pooled_ref, head_ref, tail_ref, lab_ref, W1_ref, b1_ref,
     W2_ref, b2_ref) = refs[:8]
    logits_ref, loss_ref = refs[-2:]
    i = pl.program_id(0)
    pooled = pooled_ref[...]
    hc = jnp.sum((head_ref[...] != 0).astype(jnp.int32), axis=1,
                 keepdims=True).astype(jnp.float32)
    tc = jnp.sum((tail_ref[...] != 0).astype(jnp.int32), axis=1,
                 keepdims=True).astype(jnp.float32)
    x = jnp.concatenate([pooled[:, :DIM] / hc, pooled[:, DIM:] / tc],
                        axis=1)
    h = jnp.maximum(
        jnp.dot(x, W1_ref[...], preferred_element_type=jnp.float32)
        + b1_ref[...], 0.0)
    logits = jnp.dot(h, W2_ref[...],
                     preferred_element_type=jnp.float32) + b2_ref[...]
    logits_ref[...] = logits
    m = jnp.max(logits, axis=1, keepdims=True)
    lse = jnp.log(jnp.sum(jnp.exp(logits - m), axis=1, keepdims=True)) + m
    cls = lax.broadcasted_iota(jnp.int32, logits.shape, 1)
    picked = jnp.sum(jnp.where(cls == lab_ref[...], logits, 0.0), axis=1,
                     keepdims=True)
    nll = lse - picked

    @pl.when(i == 0)
    def _():
        loss_ref[...] = jnp.zeros_like(loss_ref)

    loss_ref[...] += (jnp.sum(nll) * (1.0 / B)).reshape(1, 1)


def _mlp(pooled, head, tail, labels_col, W1, b1, W2, b2, base, prev=None):
    grid = (BC // BM,)
    in_specs = [
        pl.BlockSpec((BM, 2 * DIM), lambda i: (i, 0)),
        pl.BlockSpec((BM, L), lambda i: (i, 0)),
        pl.BlockSpec((BM, L), lambda i: (i, 0)),
        pl.BlockSpec((BM, 1), lambda i: (i, 0)),
        pl.BlockSpec((2 * DIM, DIM), lambda i: (0, 0)),
        pl.BlockSpec((1, DIM), lambda i: (0, 0)),
        pl.BlockSpec((DIM, NUM_CLASS), lambda i: (0, 0)),
        pl.BlockSpec((1, NUM_CLASS), lambda i: (0, 0)),
    ]
    args = [pooled, head, tail, labels_col, W1, b1, W2, b2]
    aliases = {}
    if prev is not None:
        in_specs.append(pl.BlockSpec((BM, NUM_CLASS), lambda i: (0, 0)))
        args.append(prev)
        aliases = {8: 0}
    return pl.pallas_call(
        _mlp_body,
        grid=grid,
        in_specs=in_specs,
        out_specs=[
            pl.BlockSpec((BM, NUM_CLASS), lambda i, base=base: (base + i, 0)),
            pl.BlockSpec((1, 1), lambda i: (0, 0)),
        ],
        out_shape=[
            jax.ShapeDtypeStruct((B, NUM_CLASS), jnp.float32),
            jax.ShapeDtypeStruct((1, 1), jnp.float32),
        ],
        input_output_aliases=aliases,
    )(*args)


def kernel(head, tail, labels, emb_table, W1, b1, W2, b2):
    head = head.astype(jnp.int32)
    tail = tail.astype(jnp.int32)
    labels_col = labels.reshape(B, 1)
    b1r = b1.reshape(1, DIM)
    b2r = b2.reshape(1, NUM_CLASS)
    idx2 = jnp.concatenate([head, tail], axis=1)

    pooled = [_sc_pool(idx2[c * BC:(c + 1) * BC], emb_table)
              for c in range(CH)]
    logits = None
    loss = jnp.float32(0.0)
    for c in range(CH):
        sl = slice(c * BC, (c + 1) * BC)
        logits, loss_c = _mlp(pooled[c], head[sl], tail[sl], labels_col[sl],
                              W1, b1r, W2, b2r, base=c * (BC // BM),
                              prev=logits)
        loss = loss + loss_c[0, 0]
    return logits, loss


# speedup vs baseline: 1.0636x; 1.0074x over previous
import functools

import jax
import jax.numpy as jnp
from jax import lax
from jax.experimental import pallas as pl
from jax.experimental.pallas import tpu as pltpu
from jax.experimental.pallas import tpu_sc as plsc

B = 16384
L = 50
DIM = 128
NUM_CLASS = 1000

NC = 2
NS = 16
NW = NC * NS

NBUF = 4
FLUSH_T = 4

BM = 2048
CH = 1
BC = B // CH


def _sc_pool(idx2, table):
    g_per_w = BC // NW
    nt = g_per_w // NBUF
    mesh = plsc.VectorSubcoreMesh(core_axis_name="c", subcore_axis_name="s")

    @functools.partial(
        pl.kernel,
        out_type=jax.ShapeDtypeStruct((BC, 2 * DIM), jnp.float32),
        mesh=mesh,
        scratch_types=[
            pltpu.VMEM((g_per_w, 2 * L), jnp.int32),
            pltpu.VMEM((NBUF, 2 * L, DIM), jnp.float32),
            pltpu.VMEM((NBUF * FLUSH_T, 2 * DIM), jnp.float32),
            pltpu.SemaphoreType.DMA((NBUF,)),
        ],
    )
    def sc_pool(idx_hbm, table_hbm, out_hbm, idx_v, bufs, out_v, sems):
        wid = lax.axis_index("s") * NC + lax.axis_index("c")
        g0 = wid * g_per_w
        row0 = wid * g_per_w

        pltpu.sync_copy(idx_hbm.at[pl.ds(g0, g_per_w)], idx_v)

        def fire(g, b):
            pltpu.make_async_copy(
                table_hbm.at[idx_v.at[g]], bufs.at[b], sems.at[b]
            ).start()

        def drain(g, b):
            pltpu.make_async_copy(
                table_hbm.at[idx_v.at[g]], bufs.at[b], sems.at[b]
            ).wait()

        for b in range(NBUF):
            fire(b, b)

        def outer(t, carry):
            for b in range(NBUF):
                g = t * NBUF + b
                drain(g, b)
                lr = (t % FLUSH_T) * NBUF + b
                for j in range(2):
                    def body(r, accs):
                        row = j * L + 2 * r
                        return tuple(
                            accs[k]
                            + bufs[b, row, pl.ds(k * 16, 16)]
                            + bufs[b, row + 1, pl.ds(k * 16, 16)]
                            for k in range(8)
                        )

                    accs = lax.fori_loop(
                        0, L // 2, body,
                        tuple(jnp.zeros((16,), jnp.float32) for _ in range(8)),
                    )
                    for k in range(8):
                        out_v[lr, pl.ds(j * DIM + k * 16, 16)] = accs[k]

                @pl.when(t + 1 < nt)
                def _():
                    fire(g + NBUF, b)

            @pl.when(t % FLUSH_T == FLUSH_T - 1)
            def _():
                base = pl.multiple_of(
                    row0 + (t // FLUSH_T) * (NBUF * FLUSH_T), NBUF * FLUSH_T)
                pltpu.sync_copy(
                    out_v, out_hbm.at[pl.ds(base, NBUF * FLUSH_T)]
                )

            return carry

        lax.fori_loop(0, nt, outer, 0)

    return sc_pool(idx2, table)


def _mlp_body(*refs):
    (pooled_ref, head_ref, tail_ref, lab_ref, W1_ref, b1_ref,
     W2_ref, b2_ref) = refs[:8]
    logits_ref, loss_ref = refs[-2:]
    i = pl.program_id(0)
    pooled = pooled_ref[...]
    hc = jnp.sum((head_ref[...] != 0).astype(jnp.int32), axis=1,
                 keepdims=True).astype(jnp.float32)
    tc = jnp.sum((tail_ref[...] != 0).astype(jnp.int32), axis=1,
                 keepdims=True).astype(jnp.float32)
    x = jnp.concatenate([pooled[:, :DIM] / hc, pooled[:, DIM:] / tc],
                        axis=1)
    h = jnp.maximum(
        jnp.dot(x, W1_ref[...], preferred_element_type=jnp.float32)
        + b1_ref[...], 0.0)
    logits = jnp.dot(h, W2_ref[...],
                     preferred_element_type=jnp.float32) + b2_ref[...]
    logits_ref[...] = logits
    m = jnp.max(logits, axis=1, keepdims=True)
    lse = jnp.log(jnp.sum(jnp.exp(logits - m), axis=1, keepdims=True)) + m
    cls = lax.broadcasted_iota(jnp.int32, logits.shape, 1)
    picked = jnp.sum(jnp.where(cls == lab_ref[...], logits, 0.0), axis=1,
                     keepdims=True)
    nll = lse - picked

    @pl.when(i == 0)
    def _():
        loss_ref[...] = jnp.zeros_like(loss_ref)

    loss_ref[...] += (jnp.sum(nll) * (1.0 / B)).reshape(1, 1)


def _mlp(pooled, head, tail, labels_col, W1, b1, W2, b2, base, prev=None):
    grid = (BC // BM,)
    in_specs = [
        pl.BlockSpec((BM, 2 * DIM), lambda i: (i, 0)),
        pl.BlockSpec((BM, L), lambda i: (i, 0)),
        pl.BlockSpec((BM, L), lambda i: (i, 0)),
        pl.BlockSpec((BM, 1), lambda i: (i, 0)),
        pl.BlockSpec((2 * DIM, DIM), lambda i: (0, 0)),
        pl.BlockSpec((1, DIM), lambda i: (0, 0)),
        pl.BlockSpec((DIM, NUM_CLASS), lambda i: (0, 0)),
        pl.BlockSpec((1, NUM_CLASS), lambda i: (0, 0)),
    ]
    args = [pooled, head, tail, labels_col, W1, b1, W2, b2]
    aliases = {}
    if prev is not None:
        in_specs.append(pl.BlockSpec((BM, NUM_CLASS), lambda i: (0, 0)))
        args.append(prev)
        aliases = {8: 0}
    return pl.pallas_call(
        _mlp_body,
        grid=grid,
        in_specs=in_specs,
        out_specs=[
            pl.BlockSpec((BM, NUM_CLASS), lambda i, base=base: (base + i, 0)),
            pl.BlockSpec((1, 1), lambda i: (0, 0)),
        ],
        out_shape=[
            jax.ShapeDtypeStruct((B, NUM_CLASS), jnp.float32),
            jax.ShapeDtypeStruct((1, 1), jnp.float32),
        ],
        input_output_aliases=aliases,
    )(*args)


def kernel(head, tail, labels, emb_table, W1, b1, W2, b2):
    head = head.astype(jnp.int32)
    tail = tail.astype(jnp.int32)
    labels_col = labels.reshape(B, 1)
    b1r = b1.reshape(1, DIM)
    b2r = b2.reshape(1, NUM_CLASS)
    idx2 = jnp.concatenate([head, tail], axis=1)

    pooled = [_sc_pool(idx2[c * BC:(c + 1) * BC], emb_table)
              for c in range(CH)]
    logits = None
    loss = jnp.float32(0.0)
    for c in range(CH):
        sl = slice(c * BC, (c + 1) * BC)
        logits, loss_c = _mlp(pooled[c], head[sl], tail[sl], labels_col[sl],
                              W1, b1r, W2, b2r, base=c * (BC // BM),
                              prev=logits)
        loss = loss + loss_c[0, 0]
    return logits, loss
